# baseline (speedup 1.0000x reference)
import functools

import numpy as np
import jax
import jax.numpy as jnp
from jax import lax
from jax.experimental import pallas as pl
from jax.experimental.pallas import tpu as pltpu
from jax.experimental.pallas import tpu_sc as plsc

_B, _N, _D = 64, 576, 768
_NKEEP = 144
_MASK_RATIO_CHECK = _NKEEP == int(_N * (1.0 - 0.75))

_NC, _NS = 2, 16
_NW = _NC * _NS
_ROWS = _B * _NKEEP
_RPW = _ROWS // _NW
_CHUNK = 24
_NCHUNK = _RPW // _CHUNK
_NBUF = 6

_consts_cache = None


def _threefry_noise(shape):
    n = int(np.prod(shape))
    with np.errstate(over="ignore"):
        x = [np.zeros(n, np.uint32), np.arange(n, dtype=np.uint32)]
        k0, k1 = np.uint32(0), np.uint32(42)
        rotations = [(13, 15, 26, 6), (17, 29, 16, 24)]
        ks = [k0, k1, np.uint32(k0 ^ k1 ^ np.uint32(0x1BD11BDA))]

        def rotl(v, d):
            return (v << np.uint32(d)) | (v >> np.uint32(32 - d))

        x[0] = x[0] + ks[0]
        x[1] = x[1] + ks[1]
        for i in range(5):
            for r in rotations[i % 2]:
                x[0] = x[0] + x[1]
                x[1] = rotl(x[1], r)
                x[1] = x[1] ^ x[0]
            x[0] = x[0] + ks[(i + 1) % 3]
            x[1] = x[1] + ks[(i + 2) % 3] + np.uint32(i + 1)
        bits = x[0] ^ x[1]
    f = ((bits >> np.uint32(9)) | np.uint32(0x3F800000)).view(np.float32)
    f = f - np.float32(1.0)
    return np.maximum(np.float32(0.0), f).reshape(shape)


def _constants():
    global _consts_cache
    if _consts_cache is None:
        noise = _threefry_noise((_B, _N))
        sorted_idx = np.argsort(noise, axis=1, kind="stable").astype(np.int32)
        pos_idx = np.argsort(sorted_idx, axis=1, kind="stable").astype(np.int32)
        mask = (pos_idx >= _NKEEP).astype(np.float32)
        keep = sorted_idx[:, :_NKEEP].astype(np.int32)
        flat_idx = (keep + np.arange(_B, dtype=np.int32)[:, None] * _N).reshape(-1)
        _consts_cache = (flat_idx, mask, pos_idx, sorted_idx)
    return _consts_cache


def _run_sc(x_flat, idx):
    mesh = plsc.VectorSubcoreMesh(core_axis_name="c", subcore_axis_name="s")

    @functools.partial(
        pl.kernel,
        mesh=mesh,
        out_type=jax.ShapeDtypeStruct((_ROWS, _D), jnp.float32),
        scratch_types=(
            [pltpu.VMEM((_RPW,), jnp.int32)]
            + [pltpu.VMEM((_CHUNK, _D), jnp.float32)] * _NBUF
            + [pltpu.SemaphoreType.DMA] * (2 * _NBUF)
        ),
    )
    def k(x_hbm, idx_hbm, out_hbm, idx_v, *scratch):
        bufs = list(scratch[:_NBUF])
        gsem = list(scratch[_NBUF:2 * _NBUF])
        ssem = list(scratch[2 * _NBUF:3 * _NBUF])
        wid = lax.axis_index("s") * _NC + lax.axis_index("c")
        base = wid * _RPW
        pltpu.sync_copy(idx_hbm.at[pl.ds(base, _RPW)], idx_v)
        gathers = [None] * _NBUF
        scatters = [None] * _NBUF
        for c in range(min(_NBUF, _NCHUNK)):
            gathers[c] = pltpu.async_copy(
                x_hbm.at[idx_v.at[pl.ds(c * _CHUNK, _CHUNK)]], bufs[c], gsem[c]
            )
        for c in range(_NCHUNK):
            cur = c % _NBUF
            gathers[cur].wait()
            scatters[cur] = pltpu.async_copy(
                bufs[cur], out_hbm.at[pl.ds(base + c * _CHUNK, _CHUNK)], ssem[cur]
            )
            p = c - 1
            nc = p + _NBUF
            if p >= 0 and nc < _NCHUNK:
                prev = p % _NBUF
                scatters[prev].wait()
                gathers[prev] = pltpu.async_copy(
                    x_hbm.at[idx_v.at[pl.ds(nc * _CHUNK, _CHUNK)]], bufs[prev], gsem[prev]
                )
        for c in range(max(0, _NCHUNK - _NBUF), _NCHUNK):
            scatters[c % _NBUF].wait()

    return k(x_flat, idx)


def kernel(x):
    flat_idx, mask, pos_idx, sorted_idx = _constants()
    x_flat = x.reshape(_B * _N, _D)
    out = _run_sc(x_flat, jnp.asarray(flat_idx))
    x_keep = out.reshape(_B, _NKEEP, _D)
    return (
        x_keep,
        jnp.asarray(mask),
        jnp.asarray(pos_idx),
        jnp.asarray(sorted_idx),
    )


# speedup vs baseline: 1.0024x; 1.0024x over previous
import functools

import numpy as np
import jax
import jax.numpy as jnp
from jax import lax
from jax.experimental import pallas as pl
from jax.experimental.pallas import tpu as pltpu
from jax.experimental.pallas import tpu_sc as plsc

_B, _N, _D = 64, 576, 768
_NKEEP = 144

_NC, _NS = 2, 16
_NW = _NC * _NS
_ROWS = _B * _NKEEP
_RPW = _ROWS // _NW
_CHUNK = 24
_NCHUNK = _RPW // _CHUNK
_NBUF = 6

_consts_cache = None


def _threefry_noise(shape):
    n = int(np.prod(shape))
    with np.errstate(over="ignore"):
        x = [np.zeros(n, np.uint32), np.arange(n, dtype=np.uint32)]
        k0, k1 = np.uint32(0), np.uint32(42)
        rotations = [(13, 15, 26, 6), (17, 29, 16, 24)]
        ks = [k0, k1, np.uint32(k0 ^ k1 ^ np.uint32(0x1BD11BDA))]

        def rotl(v, d):
            return (v << np.uint32(d)) | (v >> np.uint32(32 - d))

        x[0] = x[0] + ks[0]
        x[1] = x[1] + ks[1]
        for i in range(5):
            for r in rotations[i % 2]:
                x[0] = x[0] + x[1]
                x[1] = rotl(x[1], r)
                x[1] = x[1] ^ x[0]
            x[0] = x[0] + ks[(i + 1) % 3]
            x[1] = x[1] + ks[(i + 2) % 3] + np.uint32(i + 1)
        bits = x[0] ^ x[1]
    f = ((bits >> np.uint32(9)) | np.uint32(0x3F800000)).view(np.float32)
    f = f - np.float32(1.0)
    return np.maximum(np.float32(0.0), f).reshape(shape)


def _constants():
    global _consts_cache
    if _consts_cache is None:
        noise = _threefry_noise((_B, _N))
        sorted_idx = np.argsort(noise, axis=1, kind="stable").astype(np.int32)
        pos_idx = np.argsort(sorted_idx, axis=1, kind="stable").astype(np.int32)
        mask = (pos_idx >= _NKEEP).astype(np.float32)
        keep = sorted_idx[:, :_NKEEP].astype(np.int32)
        flat_idx = (keep + np.arange(_B, dtype=np.int32)[:, None] * _N).reshape(-1)
        _consts_cache = (flat_idx, mask, pos_idx, sorted_idx)
    return _consts_cache


def _run_sc(x_flat, idx):
    mesh = plsc.VectorSubcoreMesh(core_axis_name="c", subcore_axis_name="s")

    @functools.partial(
        pl.kernel,
        mesh=mesh,
        out_type=jax.ShapeDtypeStruct((_ROWS, _D), jnp.float32),
        scratch_types=(
            [pltpu.VMEM((_RPW,), jnp.int32)]
            + [pltpu.VMEM((_CHUNK, _D), jnp.float32)] * _NBUF
            + [pltpu.SemaphoreType.DMA] * (2 * _NBUF)
        ),
    )
    def k(x_hbm, idx_hbm, out_hbm, idx_v, *scratch):
        bufs = list(scratch[:_NBUF])
        gsem = list(scratch[_NBUF:2 * _NBUF])
        ssem = list(scratch[2 * _NBUF:3 * _NBUF])
        wid = lax.axis_index("s") * _NC + lax.axis_index("c")
        base = wid * _RPW
        pltpu.sync_copy(idx_hbm.at[pl.ds(base, _RPW)], idx_v)
        gathers = [None] * _NBUF
        scatters = [None] * _NBUF
        for c in range(min(_NBUF, _NCHUNK)):
            gathers[c] = pltpu.async_copy(
                x_hbm.at[idx_v.at[pl.ds(c * _CHUNK, _CHUNK)]], bufs[c], gsem[c]
            )
        for c in range(_NCHUNK):
            cur = c % _NBUF
            gathers[cur].wait()
            scatters[cur] = pltpu.async_copy(
                bufs[cur], out_hbm.at[pl.ds(base + c * _CHUNK, _CHUNK)], ssem[cur]
            )
            p = c - 1
            nc = p + _NBUF
            if p >= 0 and nc < _NCHUNK:
                prev = p % _NBUF
                scatters[prev].wait()
                gathers[prev] = pltpu.async_copy(
                    x_hbm.at[idx_v.at[pl.ds(nc * _CHUNK, _CHUNK)]], bufs[prev], gsem[prev]
                )
        for c in range(max(0, _NCHUNK - _NBUF), _NCHUNK):
            scatters[c % _NBUF].wait()

    return k(x_flat, idx)


def kernel(x):
    flat_idx, mask, pos_idx, sorted_idx = _constants()
    x_flat = x.reshape(_B * _N, _D)
    out = _run_sc(x_flat, jnp.asarray(flat_idx))
    x_keep = out.reshape(_B, _NKEEP, _D)
    return (
        x_keep,
        jnp.asarray(mask),
        jnp.asarray(pos_idx),
        jnp.asarray(sorted_idx),
    )
